# baseline (speedup 1.0000x reference)
import functools

import jax
import jax.numpy as jnp
import numpy as np
from jax import lax
from jax.experimental import pallas as pl
from jax.experimental.pallas import tpu as pltpu
from jax.experimental.pallas import tpu_sc as plsc

D_MODEL = 2048
MAX_LEN = 2048
VOCAB = 80
BATCH = 4
B_TC = 2
B_SC = BATCH - B_TC

_INFO = plsc.get_sparse_core_info()
NC, NS, L = _INFO.num_cores, _INFO.num_subcores, _INFO.num_lanes
NW = NC * NS
PPW = MAX_LEN // NW
KP = 8
NCH = PPW // KP
NSTEP = NCH * B_SC
NBUF = 3


def _pe_table_np():
    even_i = np.arange(0, D_MODEL, 2, dtype=np.float32)
    denominator = np.power(10000.0, even_i / D_MODEL)
    position = np.arange(MAX_LEN, dtype=np.float32).reshape(MAX_LEN, 1)
    even_pe = np.sin(position / denominator)
    odd_pe = np.cos(position / denominator)
    stacked = np.stack([even_pe, odd_pe], axis=2)
    return stacked.reshape(MAX_LEN, D_MODEL).astype(np.float32)


_PE = _pe_table_np()


def _sc_embed(tokens_flat, table, pe):
    mesh = plsc.VectorSubcoreMesh(core_axis_name="c", subcore_axis_name="s")

    @functools.partial(
        pl.kernel,
        mesh=mesh,
        out_type=jax.ShapeDtypeStruct((B_SC, MAX_LEN, D_MODEL), jnp.float32),
        scratch_types=[
            pltpu.VMEM((B_SC * PPW,), jnp.int32),
            pltpu.VMEM((KP, D_MODEL), jnp.float32),
            pltpu.VMEM((KP, D_MODEL), jnp.float32),
            pltpu.VMEM((KP, D_MODEL), jnp.float32),
            pltpu.VMEM((KP, D_MODEL), jnp.float32),
            pltpu.VMEM((KP, D_MODEL), jnp.float32),
            pltpu.SemaphoreType.DMA,
            pltpu.SemaphoreType.DMA,
            pltpu.SemaphoreType.DMA,
            pltpu.SemaphoreType.DMA,
            pltpu.SemaphoreType.DMA,
            pltpu.SemaphoreType.DMA,
            pltpu.SemaphoreType.DMA,
            pltpu.SemaphoreType.DMA,
        ],
    )
    def k(tok_hbm, table_hbm, pe_hbm, out_hbm, idx_v,
          g0, g1, g2, pb0, pb1,
          sg0, sg1, sg2, sp0, sp1, so0, so1, so2):
        wid = lax.axis_index("s") * NC + lax.axis_index("c")
        pos0 = wid * PPW

        G = [g0, g1, g2]
        P = [pb0, pb1]
        SG = [sg0, sg1, sg2]
        SP = [sp0, sp1]
        SO = [so0, so1, so2]

        for b in range(B_SC):
            pltpu.sync_copy(tok_hbm.at[pl.ds(b * MAX_LEN + pos0, PPW)],
                            idx_v.at[pl.ds(b * PPW, PPW)])

        def start_g(s):
            c, b = divmod(s, B_SC)
            kb = s % NBUF
            pltpu.async_copy(
                table_hbm.at[idx_v.at[pl.ds(b * PPW + c * KP, KP)]],
                G[kb], SG[kb])

        def wait_g(s):
            c, b = divmod(s, B_SC)
            kb = s % NBUF
            pltpu.make_async_copy(
                table_hbm.at[idx_v.at[pl.ds(b * PPW + c * KP, KP)]],
                G[kb], SG[kb]).wait()

        def start_pe(c):
            pltpu.async_copy(pe_hbm.at[pl.ds(pos0 + c * KP, KP)],
                             P[c % 2], SP[c % 2])

        def wait_pe(c):
            pltpu.make_async_copy(pe_hbm.at[pl.ds(pos0 + c * KP, KP)],
                                  P[c % 2], SP[c % 2]).wait()

        def add(s):
            kb = s % NBUF
            g, pbuf = G[kb], P[(s // B_SC) % 2]

            def body(i, acc):
                for r in range(KP):
                    g[r, pl.ds(i * L, L)] = (
                        g[r, pl.ds(i * L, L)] + pbuf[r, pl.ds(i * L, L)])
                return acc
            lax.fori_loop(0, D_MODEL // L, body, 0)

        def start_out(s):
            c, b = divmod(s, B_SC)
            kb = s % NBUF
            pltpu.async_copy(
                G[kb], out_hbm.at[b, pl.ds(pos0 + c * KP, KP)], SO[kb])

        def wait_out(s):
            c, b = divmod(s, B_SC)
            kb = s % NBUF
            pltpu.make_async_copy(
                G[kb], out_hbm.at[b, pl.ds(pos0 + c * KP, KP)], SO[kb]).wait()

        start_pe(0)
        start_pe(1)
        start_g(0)
        start_g(1)
        for s in range(NSTEP):
            c, b = divmod(s, B_SC)
            if b == 0:
                wait_pe(c)
            wait_g(s)
            add(s)
            start_out(s)
            if b == B_SC - 1 and c + 2 < NCH:
                start_pe(c + 2)
            if s + 2 < NSTEP:
                if s >= 1:
                    wait_out(s - 1)
                start_g(s + 2)
        wait_out(NSTEP - 3)
        wait_out(NSTEP - 2)
        wait_out(NSTEP - 1)

    return k(tokens_flat, table, pe)


def _tc_embed(tokens_slice, table, pe, pl_blk=256):
    nblk = MAX_LEN // pl_blk
    tok3 = tokens_slice.reshape(B_TC * nblk, 1, pl_blk)

    def kern(tok_ref, tab_ref, pe_ref, out_ref):
        tok = tok_ref[0, 0]
        oh = (tok[:, None] ==
              lax.broadcasted_iota(jnp.int32, (pl_blk, VOCAB), 1)
              ).astype(jnp.float32)
        emb = jax.lax.dot_general(
            oh, tab_ref[...], (((1,), (0,)), ((), ())),
            preferred_element_type=jnp.float32)
        out_ref[0] = emb + pe_ref[...]

    return pl.pallas_call(
        kern,
        grid=(nblk, B_TC),
        in_specs=[
            pl.BlockSpec((1, 1, pl_blk), lambda i, b: (b * nblk + i, 0, 0)),
            pl.BlockSpec((VOCAB, D_MODEL), lambda i, b: (0, 0)),
            pl.BlockSpec((pl_blk, D_MODEL), lambda i, b: (i, 0)),
        ],
        out_specs=pl.BlockSpec((1, pl_blk, D_MODEL), lambda i, b: (b, i, 0)),
        out_shape=jax.ShapeDtypeStruct((B_TC, MAX_LEN, D_MODEL), jnp.float32),
    )(tok3, table, pe)


def kernel(tokens, table):
    pe = jnp.asarray(_PE)
    out_tc = _tc_embed(tokens[:B_TC], table, pe)
    out_sc = _sc_embed(tokens[B_TC:].reshape(B_SC * MAX_LEN), table, pe)
    return jnp.concatenate([out_tc, out_sc], axis=0)
